# baseline (speedup 1.0000x reference)
import functools

import jax
import jax.numpy as jnp
from jax import lax
from jax.experimental import pallas as pl
from jax.experimental.pallas import tpu as pltpu
from jax.experimental.pallas import tpu_sc as plsc

_NC, _NS = 2, 16
_NW = _NC * _NS
_LANE = 16


def _build(B, T, V, D, C):
    N = B * T
    per_w = N // _NW
    n_chunks = per_w // C
    lanes_per_row = D // _LANE

    mesh = plsc.VectorSubcoreMesh(
        core_axis_name="c", subcore_axis_name="s",
        num_cores=_NC, num_subcores=_NS)

    @functools.partial(
        pl.kernel,
        mesh=mesh,
        out_type=jax.ShapeDtypeStruct((N, D), jnp.float32),
        scratch_types=[
            pltpu.VMEM((per_w,), jnp.int32),
            pltpu.VMEM((C, D), jnp.float32),
            pltpu.VMEM((C, D), jnp.float32),
            pltpu.SemaphoreType.DMA,
        ],
    )
    def k(idx_hbm, tok_hbm, pos_hbm, out_hbm, idx_v, tok_v, pos_v, sem):
        wid = lax.axis_index("s") * _NC + lax.axis_index("c")
        base = wid * per_w
        t0 = lax.rem(base, T)
        pltpu.sync_copy(idx_hbm.at[pl.ds(base, per_w)], idx_v)

        for ci in range(n_chunks):
            r0 = ci * C
            gat = pltpu.async_copy(tok_hbm.at[idx_v.at[pl.ds(r0, C)]], tok_v, sem)
            pltpu.sync_copy(pos_hbm.at[pl.ds(t0 + r0, C)], pos_v)
            gat.wait()

            def row_body(r, _):
                for j in range(lanes_per_row):
                    s = pl.ds(j * _LANE, _LANE)
                    tok_v[r, s] = tok_v[r, s] + pos_v[r, s]
                return 0

            lax.fori_loop(0, C, row_body, 0)
            pltpu.sync_copy(tok_v, out_hbm.at[pl.ds(base + r0, C)])

    return k


def kernel(idx, token_emb, pos_emb):
    B, T = idx.shape
    V, D = token_emb.shape
    idx_flat = idx.reshape(B * T).astype(jnp.int32)
    out = _build(B, T, V, D, C=32)(idx_flat, token_emb, pos_emb)
    return out.reshape(B, T, D)


# speedup vs baseline: 1.5220x; 1.5220x over previous
import functools

import jax
import jax.numpy as jnp
from jax import lax
from jax.experimental import pallas as pl
from jax.experimental.pallas import tpu as pltpu
from jax.experimental.pallas import tpu_sc as plsc

_NC, _NS = 2, 16
_NW = _NC * _NS
_LANE = 16


def _build(B, T, V, D, C):
    PT = T // _NW
    NTC = PT // C
    n_units = NTC * B
    per_w = PT * B
    lanes_per_row = D // _LANE

    mesh = plsc.VectorSubcoreMesh(
        core_axis_name="c", subcore_axis_name="s",
        num_cores=_NC, num_subcores=_NS)

    @functools.partial(
        pl.kernel,
        mesh=mesh,
        out_type=jax.ShapeDtypeStruct((B * T, D), jnp.float32),
        scratch_types=[
            pltpu.VMEM((per_w,), jnp.int32),
            pltpu.VMEM((2, C, D), jnp.float32),
            pltpu.VMEM((C, D), jnp.float32),
            pltpu.SemaphoreType.DMA,
            pltpu.SemaphoreType.DMA,
            pltpu.SemaphoreType.DMA,
            pltpu.SemaphoreType.DMA,
        ],
    )
    def k(idx_hbm, tok_hbm, pos_hbm, out_hbm, idx_v, tok_v, pos_v,
          gsem0, gsem1, osem0, osem1):
        wid = lax.axis_index("s") * _NC + lax.axis_index("c")
        t0 = wid * PT
        pltpu.sync_copy(idx_hbm.at[pl.ds(wid * per_w, per_w)], idx_v)

        gsems = (gsem0, gsem1)
        osems = (osem0, osem1)

        def unit_coords(u):
            tc, b = divmod(u, B)
            return tc, b

        def issue_gather(u):
            pb = u % 2
            return pltpu.async_copy(
                tok_hbm.at[idx_v.at[pl.ds(u * C, C)]], tok_v.at[pb], gsems[pb])

        def issue_out(u):
            tc, b = unit_coords(u)
            row0 = b * T + t0 + tc * C
            pb = u % 2
            return pltpu.async_copy(tok_v.at[pb], out_hbm.at[pl.ds(row0, C)],
                                    osems[pb])

        gathers = [None] * n_units
        outs = [None] * n_units
        gathers[0] = issue_gather(0)

        for u in range(n_units):
            pb = u % 2
            tc, b = unit_coords(u)
            if b == 0:
                pltpu.sync_copy(pos_hbm.at[pl.ds(t0 + tc * C, C)], pos_v)
            gathers[u].wait()
            if u + 1 < n_units:
                if u >= 1:
                    outs[u - 1].wait()
                gathers[u + 1] = issue_gather(u + 1)

            def row_body(r, _):
                for j in range(lanes_per_row):
                    s = pl.ds(j * _LANE, _LANE)
                    plsc.addupdate(tok_v.at[pb, r, s], pos_v[r, s])
                return 0

            lax.fori_loop(0, C, row_body, 0)
            outs[u] = issue_out(u)

        outs[n_units - 2].wait()
        outs[n_units - 1].wait()

    return k


def kernel(idx, token_emb, pos_emb):
    B, T = idx.shape
    V, D = token_emb.shape
    C = 32
    PT = T // _NW
    idx_re = idx.astype(jnp.int32).reshape(B, _NW, PT // C, C)
    idx_re = jnp.transpose(idx_re, (1, 2, 0, 3)).reshape(-1)
    out = _build(B, T, V, D, C)(idx_re, token_emb, pos_emb)
    return out.reshape(B, T, D)
